# Initial kernel scaffold; baseline (speedup 1.0000x reference)
import jax
import jax.numpy as jnp
from jax.experimental import pallas as pl


def kernel(x, w1, b1, w2, b2, w3, b3, fc1_w, fc1_b, fc2_w, fc2_b, fc3_w, fc3_b):
    raise NotImplementedError("write your pallas kernel here")


# speedup vs baseline: 5.6273x; 5.6273x over previous
import functools

import jax
import jax.numpy as jnp
from jax.experimental import pallas as pl
from jax.experimental.pallas import tpu as pltpu

_NB = 4


def _conv_tower_kernel(xq_ref, w1_ref, bt1_ref, w2_ref, bt2_ref,
                       w3_ref, bt3_ref, o_ref):
    f32 = jnp.float32
    bf16 = jnp.bfloat16
    Ga = xq_ref[...]
    nb = Ga.shape[0]

    z16 = jnp.zeros((nb, 52, 1, 16), bf16)
    nxt = jnp.concatenate([Ga[:, :, 1:, 0:16], z16], axis=2)
    W6 = jnp.concatenate([Ga, nxt], axis=-1)
    A1 = jnp.concatenate([W6[:, 0:50], W6[:, 1:51], W6[:, 2:52]],
                         axis=-1)
    A1 = A1.reshape(nb * 650, 144)
    h1 = jnp.dot(A1, w1_ref[...], preferred_element_type=f32) + bt1_ref[...]
    h1 = jnp.maximum(h1, 0.0).astype(bf16).reshape(nb, 50, 13, 256)

    E0 = h1[..., 0:64].reshape(nb, 50, 13, 1, 64)
    E1 = h1[..., 64:128].reshape(nb, 50, 13, 1, 64)
    E2 = h1[..., 128:192].reshape(nb, 50, 13, 1, 64)
    E3 = h1[..., 192:256].reshape(nb, 50, 13, 1, 64)
    X0 = jnp.concatenate([E0, E2], axis=3).reshape(nb, 50, 26, 64)[:, :, 0:25]
    X1 = jnp.concatenate([E1, E3], axis=3).reshape(nb, 50, 26, 64)[:, :, 0:25]
    z64 = jnp.zeros((nb, 50, 1, 64), bf16)
    spL = jnp.concatenate([z64, X1], axis=2)
    spR = jnp.concatenate([X0, z64], axis=2)
    sp = jnp.concatenate([spL, spR], axis=-1)
    zh = jnp.zeros((nb, 1, 26, 128), bf16)
    sph = jnp.concatenate([zh, sp, zh], axis=1)

    A2 = jnp.concatenate(
        [sph[:, dy:dy + 50, po:po + 25, :] for dy in range(3) for po in range(2)],
        axis=-1)
    A2 = A2.reshape(nb * 1250, 768)
    h2 = jnp.dot(A2, w2_ref[...], preferred_element_type=f32) + bt2_ref[...]
    h2 = jnp.maximum(h2, 0.0).astype(bf16)

    pooled = jnp.maximum(h2[:, 0:128], h2[:, 128:256]).reshape(nb, 50, 25, 128)

    zw = jnp.zeros((nb, 50, 1, 128), bf16)
    pw = jnp.concatenate([zw, pooled, zw], axis=2)
    zh3 = jnp.zeros((nb, 1, 27, 128), bf16)
    ph = jnp.concatenate([zh3, pw, zh3], axis=1)
    WR = jnp.concatenate([ph[:, :, wp, :] for wp in range(27)],
                         axis=-1)
    A3 = jnp.concatenate([WR[:, 0:50], WR[:, 1:51], WR[:, 2:52]],
                         axis=-1)
    A3 = A3.reshape(nb * 50, 10368)
    h3 = jnp.dot(A3, w3_ref[...], preferred_element_type=f32) + bt3_ref[...]
    h3 = jnp.maximum(h3, 0.0)
    o_ref[...] = h3.astype(bf16).reshape(nb, 50, 50)


def _fc_kernel(f_ref, w1_ref, b1_ref, w2_ref, b2_ref, w3_ref, b3_ref, o_ref):
    f32 = jnp.float32
    h = jnp.dot(f_ref[...], w1_ref[...], preferred_element_type=f32) + b1_ref[...]
    h = jnp.maximum(h, 0.0).astype(jnp.bfloat16)
    h = jnp.dot(h, w2_ref[...], preferred_element_type=f32) + b2_ref[...]
    h = jnp.maximum(h, 0.0).astype(jnp.bfloat16)
    h = jnp.dot(h, w3_ref[...], preferred_element_type=f32) + b3_ref[...]
    o_ref[...] = h


def _build_b1(w1):
    w1r = w1.reshape(3, 3, 8, 64)
    wp = jnp.arange(6)[:, None]
    w4 = jnp.arange(4)[None, :]
    dx = wp - w4
    mask = ((dx >= 0) & (dx <= 2)).astype(w1.dtype)
    g = w1r[:, jnp.clip(dx, 0, 2), :, :]
    g = g * mask[None, :, :, None, None]
    return g.transpose(0, 1, 3, 2, 4).reshape(144, 256)


def _build_b2(w2):
    w2r = w2.reshape(3, 3, 64, 128)
    po = jnp.arange(2)[:, None, None]
    hf = jnp.arange(2)[None, :, None]
    w01 = jnp.arange(2)[None, None, :]
    dx = 2 * po + hf - w01
    mask = ((dx >= 0) & (dx <= 2)).astype(w2.dtype)
    g = w2r[:, jnp.clip(dx, 0, 2), :, :]
    g = g * mask[None, :, :, :, None, None]
    return g.transpose(0, 1, 2, 4, 3, 5).reshape(768, 256)


def _build_b3(w3):
    w3r = w3.reshape(3, 3, 128, 2)
    wp = jnp.arange(27)[:, None]
    w = jnp.arange(25)[None, :]
    dx = wp - w
    mask = ((dx >= 0) & (dx <= 2)).astype(w3.dtype)
    g = w3r[:, jnp.clip(dx, 0, 2), :, :]
    g = g * mask[None, :, :, None, None]
    return g.transpose(0, 1, 3, 2, 4).reshape(10368, 50)


def kernel(x, w1, b1, w2, b2, w3, b3, fc1_w, fc1_b, fc2_w, fc2_b, fc3_w, fc3_b):
    x = x.reshape(-1, 50, 50, 6)
    n = x.shape[0]
    nblk = n // _NB

    xp = jnp.pad(x, ((0, 0), (1, 1), (1, 1), (0, 2))).astype(jnp.bfloat16)
    xq = xp.reshape(n, 52, 13, 32)

    B1 = _build_b1(w1)
    B2 = _build_b2(w2)
    B3 = _build_b3(w3)
    bt1 = jnp.tile(b1, (1, 4))
    bt2 = jnp.tile(b2, (1, 2))
    bt3 = jnp.tile(b3, (1, 25))

    feat = pl.pallas_call(
        _conv_tower_kernel,
        out_shape=jax.ShapeDtypeStruct((n, 50, 50), jnp.bfloat16),
        grid=(nblk,),
        in_specs=[
            pl.BlockSpec((_NB, 52, 13, 32), lambda i: (i, 0, 0, 0)),
            pl.BlockSpec((144, 256), lambda i: (0, 0)),
            pl.BlockSpec((1, 256), lambda i: (0, 0)),
            pl.BlockSpec((768, 256), lambda i: (0, 0)),
            pl.BlockSpec((1, 256), lambda i: (0, 0)),
            pl.BlockSpec((10368, 50), lambda i: (0, 0)),
            pl.BlockSpec((1, 50), lambda i: (0, 0)),
        ],
        out_specs=pl.BlockSpec((_NB, 50, 50), lambda i: (i, 0, 0)),
        compiler_params=pltpu.CompilerParams(
            dimension_semantics=("parallel",)),
    )(xq, B1, bt1, B2, bt2, B3, bt3)

    fc1_wp = fc1_w.reshape(2, 50, 25, 1280).transpose(1, 2, 0, 3).reshape(2500, 1280)
    feat2 = feat.reshape(n, 2500)

    mt = min(128, n)
    out = pl.pallas_call(
        _fc_kernel,
        out_shape=jax.ShapeDtypeStruct((n, 128), jnp.float32),
        grid=(n // mt,),
        in_specs=[
            pl.BlockSpec((mt, 2500), lambda i: (i, 0)),
            pl.BlockSpec((2500, 1280), lambda i: (0, 0)),
            pl.BlockSpec((1, 1280), lambda i: (0, 0)),
            pl.BlockSpec((1280, 768), lambda i: (0, 0)),
            pl.BlockSpec((1, 768), lambda i: (0, 0)),
            pl.BlockSpec((768, 128), lambda i: (0, 0)),
            pl.BlockSpec((1, 128), lambda i: (0, 0)),
        ],
        out_specs=pl.BlockSpec((mt, 128), lambda i: (i, 0)),
        compiler_params=pltpu.CompilerParams(
            dimension_semantics=("parallel",)),
    )(feat2, fc1_wp, fc1_b, fc2_w, fc2_b, fc3_w, fc3_b)

    return out[:, :120].reshape(-1, 60, 2)


# speedup vs baseline: 11.1426x; 1.9801x over previous
import functools

import jax
import jax.numpy as jnp
from jax.experimental import pallas as pl
from jax.experimental.pallas import tpu as pltpu

_NB = 8


def _conv_tower_kernel(xs_ref, w1_ref, bt1_ref, w2_ref, bt2_ref,
                       w3_ref, bt3_ref, o_ref):
    f32 = jnp.float32
    bf16 = jnp.bfloat16
    S = xs_ref[...]
    nb = S.shape[2]

    z16 = jnp.zeros((52, 1, nb, 16), bf16)
    Sn = jnp.concatenate([S[:, 1:26], z16], axis=1)
    W4 = jnp.concatenate([S, Sn], axis=-1)
    A1 = jnp.concatenate([W4[0:50], W4[1:51], W4[2:52]],
                         axis=-1)
    A1 = A1.reshape(1300 * nb, 96)
    h1 = jnp.dot(A1, w1_ref[...], preferred_element_type=f32) + bt1_ref[...]
    h1 = jnp.maximum(h1, 0.0).astype(bf16).reshape(50, 26, nb, 128)
    z64 = jnp.zeros((50, 1, nb, 64), bf16)
    p0m = jnp.concatenate([z64, h1[:, 0:1, :, 64:128]], axis=-1)
    p25m = jnp.concatenate([h1[:, 25:26, :, 0:64], z64], axis=-1)
    sp = jnp.concatenate([p0m, h1[:, 1:25], p25m], axis=1)
    zh = jnp.zeros((1, 26, nb, 128), bf16)
    sph = jnp.concatenate([zh, sp, zh], axis=0)

    A2 = jnp.concatenate(
        [sph[dy:dy + 50, po:po + 25] for dy in range(3) for po in range(2)],
        axis=-1)
    A2 = A2.reshape(1250 * nb, 768)
    h2 = jnp.dot(A2, w2_ref[...], preferred_element_type=f32) + bt2_ref[...]
    h2 = jnp.maximum(h2, 0.0).astype(bf16)

    pooled = jnp.maximum(h2[:, 0:128], h2[:, 128:256]).reshape(50, 25, nb, 128)

    zw = jnp.zeros((50, 1, nb, 128), bf16)
    pw = jnp.concatenate([zw, pooled, zw], axis=1)
    zh3 = jnp.zeros((1, 27, nb, 128), bf16)
    ph = jnp.concatenate([zh3, pw, zh3], axis=0)
    A3 = jnp.concatenate(
        [ph[dy:dy + 50, wp] for dy in range(3) for wp in range(27)],
        axis=-1)
    A3 = A3.reshape(50 * nb, 10368)
    h3 = jnp.dot(A3, w3_ref[...], preferred_element_type=f32) + bt3_ref[...]
    h3 = jnp.maximum(h3, 0.0)
    o_ref[...] = h3.astype(bf16).reshape(50, nb, 50)


def _fc_kernel(f_ref, w1_ref, b1_ref, w2_ref, b2_ref, w3_ref, b3_ref, o_ref):
    f32 = jnp.float32
    h = jnp.dot(f_ref[...], w1_ref[...], preferred_element_type=f32) + b1_ref[...]
    h = jnp.maximum(h, 0.0).astype(jnp.bfloat16)
    h = jnp.dot(h, w2_ref[...], preferred_element_type=f32) + b2_ref[...]
    h = jnp.maximum(h, 0.0).astype(jnp.bfloat16)
    h = jnp.dot(h, w3_ref[...], preferred_element_type=f32) + b3_ref[...]
    o_ref[...] = h


def _build_b1(w1):
    w1r = w1.reshape(3, 3, 8, 64)
    wl = jnp.arange(4)[:, None]
    wloc = jnp.arange(2)[None, :]
    dx = wl - wloc
    mask = ((dx >= 0) & (dx <= 2)).astype(w1.dtype)
    g = w1r[:, jnp.clip(dx, 0, 2), :, :]
    g = g * mask[None, :, :, None, None]
    return g.transpose(0, 1, 3, 2, 4).reshape(96, 128)


def _build_b2(w2):
    w2r = w2.reshape(3, 3, 64, 128)
    po = jnp.arange(2)[:, None, None]
    hf = jnp.arange(2)[None, :, None]
    w01 = jnp.arange(2)[None, None, :]
    dx = 2 * po + hf - w01
    mask = ((dx >= 0) & (dx <= 2)).astype(w2.dtype)
    g = w2r[:, jnp.clip(dx, 0, 2), :, :]
    g = g * mask[None, :, :, :, None, None]
    return g.transpose(0, 1, 2, 4, 3, 5).reshape(768, 256)


def _build_b3(w3):
    w3r = w3.reshape(3, 3, 128, 2)
    wp = jnp.arange(27)[:, None]
    w = jnp.arange(25)[None, :]
    dx = wp - w
    mask = ((dx >= 0) & (dx <= 2)).astype(w3.dtype)
    g = w3r[:, jnp.clip(dx, 0, 2), :, :]
    g = g * mask[None, :, :, None, None]
    return g.transpose(0, 1, 3, 2, 4).reshape(10368, 50)


def kernel(x, w1, b1, w2, b2, w3, b3, fc1_w, fc1_b, fc2_w, fc2_b, fc3_w, fc3_b):
    x = x.reshape(-1, 50, 50, 6)
    n = x.shape[0]
    nblk = n // _NB

    xp = jnp.pad(x, ((0, 0), (1, 1), (2, 1), (0, 2))).astype(jnp.bfloat16)
    xs = xp[:, :, 0:52, :].reshape(n, 52, 26, 16).transpose(1, 2, 0, 3)

    B1 = _build_b1(w1)
    B2 = _build_b2(w2)
    B3 = _build_b3(w3)
    bt1 = jnp.tile(b1, (1, 2))
    bt2 = jnp.tile(b2, (1, 2))
    bt3 = jnp.tile(b3, (1, 25))

    feat = pl.pallas_call(
        _conv_tower_kernel,
        out_shape=jax.ShapeDtypeStruct((50, n, 50), jnp.bfloat16),
        grid=(nblk,),
        in_specs=[
            pl.BlockSpec((52, 26, _NB, 16), lambda i: (0, 0, i, 0)),
            pl.BlockSpec((96, 128), lambda i: (0, 0)),
            pl.BlockSpec((1, 128), lambda i: (0, 0)),
            pl.BlockSpec((768, 256), lambda i: (0, 0)),
            pl.BlockSpec((1, 256), lambda i: (0, 0)),
            pl.BlockSpec((10368, 50), lambda i: (0, 0)),
            pl.BlockSpec((1, 50), lambda i: (0, 0)),
        ],
        out_specs=pl.BlockSpec((50, _NB, 50), lambda i: (0, i, 0)),
        compiler_params=pltpu.CompilerParams(
            dimension_semantics=("parallel",)),
    )(xs, B1, bt1, B2, bt2, B3, bt3)

    fc1_wp = fc1_w.reshape(2, 50, 25, 1280).transpose(1, 2, 0, 3).reshape(2500, 1280)
    feat2 = feat.transpose(1, 0, 2).reshape(n, 2500)

    mt = min(128, n)
    out = pl.pallas_call(
        _fc_kernel,
        out_shape=jax.ShapeDtypeStruct((n, 128), jnp.float32),
        grid=(n // mt,),
        in_specs=[
            pl.BlockSpec((mt, 2500), lambda i: (i, 0)),
            pl.BlockSpec((2500, 1280), lambda i: (0, 0)),
            pl.BlockSpec((1, 1280), lambda i: (0, 0)),
            pl.BlockSpec((1280, 768), lambda i: (0, 0)),
            pl.BlockSpec((1, 768), lambda i: (0, 0)),
            pl.BlockSpec((768, 128), lambda i: (0, 0)),
            pl.BlockSpec((1, 128), lambda i: (0, 0)),
        ],
        out_specs=pl.BlockSpec((mt, 128), lambda i: (i, 0)),
        compiler_params=pltpu.CompilerParams(
            dimension_semantics=("parallel",)),
    )(feat2, fc1_wp, fc1_b, fc2_w, fc2_b, fc3_w, fc3_b)

    return out[:, :120].reshape(-1, 60, 2)
